# baseline (speedup 1.0000x reference)
import jax
import jax.numpy as jnp
from jax import lax
from jax.experimental import pallas as pl
from jax.experimental.pallas import tpu as pltpu
from jax.experimental.pallas import tpu_sc as plsc

N = 10000
H = 32
L = 4
NC, NS = 2, 16
NW = NC * NS
C = 128
NPAD = 10240
RPS = NPAD // NS
DW = 32
PK = 128 // H
NP = N // PK
NPP = NPAD // PK

_SC_PARAMS = pltpu.CompilerParams(use_tc_tiling_on_sc=False)


def _mesh():
    return plsc.VectorSubcoreMesh(
        core_axis_name="c", subcore_axis_name="s", num_cores=NC, num_subcores=NS
    )


def _deg_call(dst_r, ones16, zeros16):
    nchunk = dst_r.shape[1]

    NBD = 4
    assert nchunk % NBD == 0

    def body(dst_hbm, ones_hbm, zeros_hbm, out_hbm, dst_v, ones_v, acc,
             *dsems):
        ci = lax.axis_index("c")
        si = lax.axis_index("s")
        wid = si * NC + ci
        rows = pl.ds(si * RPS, RPS)
        pltpu.sync_copy(zeros_hbm, acc.at[rows, :])
        pltpu.sync_copy(dst_hbm.at[wid], dst_v)
        pltpu.sync_copy(ones_hbm, ones_v)
        plsc.subcore_barrier()

        def scat(i, b):
            pltpu.async_copy(ones_v, acc.at[dst_v.at[i]], dsems[b], add=True)

        def wscat(i, b):
            pltpu.make_async_copy(ones_v, acc.at[dst_v.at[i]], dsems[b]).wait()

        def outer(j, carry):
            for b in range(NBD):
                i = j * NBD + b

                @pl.when(i >= NBD)
                def _():
                    wscat(i - NBD, b)

                scat(i, b)
            return carry

        lax.fori_loop(0, nchunk // NBD, outer, 0)
        for i in range(nchunk - NBD, nchunk):
            wscat(i, i % NBD)
        plsc.subcore_barrier()
        pltpu.sync_copy(acc.at[rows, :], out_hbm.at[ci, rows, :])

    fn = pl.kernel(
        body,
        out_type=jax.ShapeDtypeStruct((NC, NPAD, DW), jnp.float32),
        mesh=_mesh(),
        compiler_params=_SC_PARAMS,
        scratch_types=[
            pltpu.VMEM((nchunk, C), jnp.int32),
            pltpu.VMEM((C, DW), jnp.float32),
            pltpu.VMEM_SHARED((NPAD, DW), jnp.float32),
        ] + [pltpu.SemaphoreType.DMA for _ in range(NBD)],
    )
    return fn(dst_r, ones16, zeros16)


def _agg_call(src_r, dst_r, table, zeros32):
    nchunk = src_r.shape[1]
    NB = 4
    assert nchunk > 2 * NB

    def body(src_hbm, dst_hbm, table_hbm, zeros_hbm, out_hbm,
             src_v, dst_v, *rest):
        bufs = rest[:NB]
        acc = rest[NB]
        gsems = rest[NB + 1:2 * NB + 1]
        ssems = rest[2 * NB + 1:3 * NB + 1]
        ci = lax.axis_index("c")
        si = lax.axis_index("s")
        wid = si * NC + ci
        rows = pl.ds(si * RPS, RPS)
        pltpu.sync_copy(zeros_hbm, acc.at[rows, :])
        pltpu.sync_copy(src_hbm.at[wid], src_v)
        pltpu.sync_copy(dst_hbm.at[wid], dst_v)
        plsc.subcore_barrier()

        def gather(i, b):
            pltpu.async_copy(table_hbm.at[src_v.at[i]], bufs[b], gsems[b])

        def wait_gather(i, b):
            pltpu.make_async_copy(
                table_hbm.at[src_v.at[i]], bufs[b], gsems[b]
            ).wait()

        def scatter(i, b):
            pltpu.async_copy(bufs[b], acc.at[dst_v.at[i]], ssems[b], add=True)

        def wait_scatter(i, b):
            pltpu.make_async_copy(
                bufs[b], acc.at[dst_v.at[i]], ssems[b]
            ).wait()

        for b in range(NB - 1):
            gather(b, b)

        nouter = (nchunk - (NB - 1)) // NB

        def outer(j, carry):
            for b in range(NB):
                i = j * NB + b
                p = (b + NB - 1) % NB

                @pl.when(i >= 1)
                def _():
                    wait_scatter(i - 1, p)

                gather(i + NB - 1, p)
                wait_gather(i, b)
                scatter(i, b)
            return carry

        lax.fori_loop(0, nouter, outer, 0)
        for i in range(nouter * NB, nchunk):
            b = i % NB
            p = (b + NB - 1) % NB
            g = i + NB - 1
            if g < nchunk:
                wait_scatter(g - NB, p)
                gather(g, p)
            wait_gather(i, b)
            scatter(i, b)
        for i in range(nchunk - NB, nchunk):
            wait_scatter(i, i % NB)
        plsc.subcore_barrier()
        pltpu.sync_copy(acc.at[rows, :], out_hbm.at[ci, rows, :])

    fn = pl.kernel(
        body,
        out_type=jax.ShapeDtypeStruct((NC, NPAD, H), jnp.float32),
        mesh=_mesh(),
        compiler_params=_SC_PARAMS,
        scratch_types=[
            pltpu.VMEM((nchunk, C), jnp.int32),
            pltpu.VMEM((nchunk, C), jnp.int32),
        ] + [pltpu.VMEM((C, H), jnp.float32) for _ in range(NB)] + [
            pltpu.VMEM_SHARED((NPAD, H), jnp.float32),
        ] + [pltpu.SemaphoreType.DMA for _ in range(2 * NB)],
    )
    return fn(src_r, dst_r, table, zeros32)


def _pack(v):
    k = v.shape[0] // PK
    v3 = jnp.reshape(v, (k, PK, H))
    return jnp.concatenate([v3[:, i, :] for i in range(PK)], axis=1)


def _unpack(v):
    k = v.shape[0]
    parts = [v[:, i * H:(i + 1) * H] for i in range(PK)]
    return jnp.reshape(jnp.stack(parts, axis=1), (k * PK, H))


def _prelude_call(degp, x, w0):
    def body(degp_ref, x_ref, w0_ref, disp_ref, hw_ref):
        dsum = degp_ref[0] + degp_ref[1]
        disp = lax.rsqrt(dsum + 1.0)[0:NP, :]
        hw = jnp.dot(x_ref[...], w0_ref[...],
                     preferred_element_type=jnp.float32)
        disp_ref[...] = disp
        hw_ref[0:NP, :] = _pack(hw) * disp
        hw_ref[NP:NPP, :] = jnp.zeros((NPP - NP, PK * H), jnp.float32)

    return pl.pallas_call(
        body,
        out_shape=(
            jax.ShapeDtypeStruct((NP, PK * H), jnp.float32),
            jax.ShapeDtypeStruct((NPP, PK * H), jnp.float32),
        ),
    )(degp, x, w0)


def _fold(s):
    parts = [s[:, i * H:(i + 1) * H] for i in range(PK)]
    f = parts[0]
    for p in parts[1:]:
        f = f + p
    return jnp.concatenate([f] * PK, axis=1)


def _bn_packed(t, gp, bep, relu):
    mean = _fold(jnp.sum(t, axis=0, keepdims=True)) / N
    d = t - mean
    var = _fold(jnp.sum(d * d, axis=0, keepdims=True)) / N
    h = d * lax.rsqrt(var + 1e-5) * gp + bep
    if relu:
        h = jnp.maximum(h, 0.0)
    return h


def _layer_call(aggp, hws, disp, bp, gp, bep, wblock):
    def body(aggp_ref, hws_ref, disp_ref, bp_ref, gp_ref, bep_ref,
             wb_ref, out_ref):
        a = aggp_ref[...]
        disp = disp_ref[...]
        t = disp * (a[0, 0:NP, :] + a[1, 0:NP, :] + hws_ref[0:NP, :]) + bp_ref[...]
        h = _bn_packed(t, gp_ref[...], bep_ref[...], relu=True)
        out_ref[0:NP, :] = (
            jnp.dot(h, wb_ref[...], preferred_element_type=jnp.float32) * disp
        )
        out_ref[NP:NPP, :] = jnp.zeros((NPP - NP, PK * H), jnp.float32)

    return pl.pallas_call(
        body, out_shape=jax.ShapeDtypeStruct((NPP, PK * H), jnp.float32)
    )(aggp, hws, disp, bp, gp, bep, wblock)


def _final_call(aggp, hws, disp, bp, gp, bep):
    def body(aggp_ref, hws_ref, disp_ref, bp_ref, gp_ref, bep_ref, out_ref):
        a = aggp_ref[...]
        disp = disp_ref[...]
        t = disp * (a[0, 0:NP, :] + a[1, 0:NP, :] + hws_ref[0:NP, :]) + bp_ref[...]
        h = _bn_packed(t, gp_ref[...], bep_ref[...], relu=False)
        out_ref[0:NP, :] = h
        out_ref[NP:NPP, :] = jnp.zeros((NPP - NP, PK * H), jnp.float32)

    return pl.pallas_call(
        body, out_shape=jax.ShapeDtypeStruct((NPP, PK * H), jnp.float32)
    )(aggp, hws, disp, bp, gp, bep)


def _pad_idx(a, nw, epw, nchunk, fill):
    a = a.reshape(nw, epw)
    pad = jnp.full((nw, nchunk * C - epw), fill, jnp.int32)
    return jnp.concatenate([a, pad], axis=1).reshape(nw, nchunk, C)


def kernel(x, edge_index, batch, Ws, bs, gammas, betas):
    del batch
    E = edge_index.shape[1]
    epw = E // NW
    nchunk = (-(-epw // C) + 7) // 8 * 8
    src_r = _pad_idx(edge_index[0], NW, epw, nchunk, 0)
    dst_r = _pad_idx(edge_index[1], NW, epw, nchunk, NPAD - 1)

    ones32 = jnp.ones((C, DW), jnp.float32)
    zeros32 = jnp.zeros((RPS, H), jnp.float32)

    degp = _deg_call(dst_r, ones32, zeros32)
    degp_pk = jnp.reshape(degp, (NC, NPP, 128))
    disp, hws = _prelude_call(degp_pk, x, Ws[0])

    for l in range(L):
        tbl = jnp.reshape(hws, (NPAD, H))
        aggp = jnp.reshape(_agg_call(src_r, dst_r, tbl, zeros32),
                           (NC, NPP, PK * H))
        bp = jnp.tile(bs[l], PK).reshape(1, PK * H)
        gp = jnp.tile(gammas[l], PK).reshape(1, PK * H)
        bep = jnp.tile(betas[l], PK).reshape(1, PK * H)
        if l != L - 1:
            wblock = jnp.kron(jnp.eye(PK, dtype=jnp.float32), Ws[l + 1])
            hws = _layer_call(aggp, hws, disp, bp, gp, bep, wblock)
        else:
            outp = _final_call(aggp, hws, disp, bp, gp, bep)
    return jnp.reshape(outp, (NPAD, H))[0:N]


# speedup vs baseline: 1.9308x; 1.9308x over previous
import jax
import jax.numpy as jnp
from jax import lax
from jax.experimental import pallas as pl
from jax.experimental.pallas import tpu as pltpu
from jax.experimental.pallas import tpu_sc as plsc

N = 10000
H = 32
L = 4
NC, NS = 2, 16
NW = NC * NS
C = 128
NPAD = 10240
RPS = NPAD // NS
DW = 32
PK = 128 // H
NP = N // PK
NPP = NPAD // PK

_SC_PARAMS = pltpu.CompilerParams(use_tc_tiling_on_sc=False)


def _mesh():
    return plsc.VectorSubcoreMesh(
        core_axis_name="c", subcore_axis_name="s", num_cores=NC, num_subcores=NS
    )


def _deg_call(dst_r, ones16, zeros16):
    nchunk = dst_r.shape[1]

    NBD = 4
    assert nchunk % NBD == 0

    def body(dst_hbm, ones_hbm, zeros_hbm, out_hbm, dst_v, ones_v, acc,
             *dsems):
        ci = lax.axis_index("c")
        si = lax.axis_index("s")
        wid = si * NC + ci
        rows = pl.ds(si * RPS, RPS)
        pltpu.sync_copy(zeros_hbm, acc.at[rows, :])
        pltpu.sync_copy(dst_hbm.at[wid], dst_v)
        pltpu.sync_copy(ones_hbm, ones_v)
        plsc.subcore_barrier()

        def scat(i, b):
            pltpu.async_copy(ones_v, acc.at[dst_v.at[i]], dsems[b], add=True)

        def wscat(i, b):
            pltpu.make_async_copy(ones_v, acc.at[dst_v.at[i]], dsems[b]).wait()

        def outer(j, carry):
            for b in range(NBD):
                i = j * NBD + b

                @pl.when(i >= NBD)
                def _():
                    wscat(i - NBD, b)

                scat(i, b)
            return carry

        lax.fori_loop(0, nchunk // NBD, outer, 0)
        for i in range(nchunk - NBD, nchunk):
            wscat(i, i % NBD)
        plsc.subcore_barrier()
        pltpu.sync_copy(acc.at[rows, :], out_hbm.at[ci, rows, :])

    fn = pl.kernel(
        body,
        out_type=jax.ShapeDtypeStruct((NC, NPAD, DW), jnp.float32),
        mesh=_mesh(),
        compiler_params=_SC_PARAMS,
        scratch_types=[
            pltpu.VMEM((nchunk, C), jnp.int32),
            pltpu.VMEM((C, DW), jnp.float32),
            pltpu.VMEM_SHARED((NPAD, DW), jnp.float32),
        ] + [pltpu.SemaphoreType.DMA for _ in range(NBD)],
    )
    return fn(dst_r, ones16, zeros16)


def _agg_call(src_r, dst_r, table, zeros32):
    nchunk = src_r.shape[1]
    NB = 4
    assert nchunk > 2 * NB

    def body(src_hbm, dst_hbm, table_hbm, zeros_hbm, out_hbm,
             src_v, dst_v, *rest):
        bufs = rest[:NB]
        tbl, acc = rest[NB], rest[NB + 1]
        gsems = rest[NB + 2:2 * NB + 2]
        ssems = rest[2 * NB + 2:3 * NB + 2]
        ci = lax.axis_index("c")
        si = lax.axis_index("s")
        wid = si * NC + ci
        rows = pl.ds(si * RPS, RPS)
        pltpu.sync_copy(table_hbm.at[rows, :], tbl.at[rows, :])
        pltpu.sync_copy(zeros_hbm, acc.at[rows, :])
        pltpu.sync_copy(src_hbm.at[wid], src_v)
        pltpu.sync_copy(dst_hbm.at[wid], dst_v)
        plsc.subcore_barrier()

        def gather(i, b):
            pltpu.async_copy(tbl.at[src_v.at[i]], bufs[b], gsems[b])

        def wait_gather(i, b):
            pltpu.make_async_copy(tbl.at[src_v.at[i]], bufs[b], gsems[b]).wait()

        def scatter(i, b):
            pltpu.async_copy(bufs[b], acc.at[dst_v.at[i]], ssems[b], add=True)

        def wait_scatter(i, b):
            pltpu.make_async_copy(
                bufs[b], acc.at[dst_v.at[i]], ssems[b]
            ).wait()

        for b in range(NB - 1):
            gather(b, b)

        nouter = (nchunk - (NB - 1)) // NB

        def outer(j, carry):
            for b in range(NB):
                i = j * NB + b
                p = (b + NB - 1) % NB

                @pl.when(i >= 1)
                def _():
                    wait_scatter(i - 1, p)

                gather(i + NB - 1, p)
                wait_gather(i, b)
                scatter(i, b)
            return carry

        lax.fori_loop(0, nouter, outer, 0)
        for i in range(nouter * NB, nchunk):
            b = i % NB
            p = (b + NB - 1) % NB
            g = i + NB - 1
            if g < nchunk:
                wait_scatter(g - NB, p)
                gather(g, p)
            wait_gather(i, b)
            scatter(i, b)
        for i in range(nchunk - NB, nchunk):
            wait_scatter(i, i % NB)
        plsc.subcore_barrier()
        pltpu.sync_copy(acc.at[rows, :], out_hbm.at[ci, rows, :])

    fn = pl.kernel(
        body,
        out_type=jax.ShapeDtypeStruct((NC, NPAD, H), jnp.float32),
        mesh=_mesh(),
        compiler_params=_SC_PARAMS,
        scratch_types=[
            pltpu.VMEM((nchunk, C), jnp.int32),
            pltpu.VMEM((nchunk, C), jnp.int32),
        ] + [pltpu.VMEM((C, H), jnp.float32) for _ in range(NB)] + [
            pltpu.VMEM_SHARED((NPAD, H), jnp.float32),
            pltpu.VMEM_SHARED((NPAD, H), jnp.float32),
        ] + [pltpu.SemaphoreType.DMA for _ in range(2 * NB)],
    )
    return fn(src_r, dst_r, table, zeros32)


def _pack(v):
    k = v.shape[0] // PK
    v3 = jnp.reshape(v, (k, PK, H))
    return jnp.concatenate([v3[:, i, :] for i in range(PK)], axis=1)


def _unpack(v):
    k = v.shape[0]
    parts = [v[:, i * H:(i + 1) * H] for i in range(PK)]
    return jnp.reshape(jnp.stack(parts, axis=1), (k * PK, H))


def _prelude_call(degp, x, w0):
    def body(degp_ref, x_ref, w0_ref, disp_ref, hw_ref):
        dsum = degp_ref[0] + degp_ref[1]
        disp = lax.rsqrt(dsum + 1.0)[0:NP, :]
        hw = jnp.dot(x_ref[...], w0_ref[...],
                     preferred_element_type=jnp.float32)
        disp_ref[...] = disp
        hw_ref[0:NP, :] = _pack(hw) * disp
        hw_ref[NP:NPP, :] = jnp.zeros((NPP - NP, PK * H), jnp.float32)

    return pl.pallas_call(
        body,
        out_shape=(
            jax.ShapeDtypeStruct((NP, PK * H), jnp.float32),
            jax.ShapeDtypeStruct((NPP, PK * H), jnp.float32),
        ),
    )(degp, x, w0)


def _fold(s):
    parts = [s[:, i * H:(i + 1) * H] for i in range(PK)]
    f = parts[0]
    for p in parts[1:]:
        f = f + p
    return jnp.concatenate([f] * PK, axis=1)


def _bn_packed(t, gp, bep, relu):
    mean = _fold(jnp.sum(t, axis=0, keepdims=True)) / N
    d = t - mean
    var = _fold(jnp.sum(d * d, axis=0, keepdims=True)) / N
    h = d * lax.rsqrt(var + 1e-5) * gp + bep
    if relu:
        h = jnp.maximum(h, 0.0)
    return h


def _layer_call(aggp, hws, disp, bp, gp, bep, wblock):
    def body(aggp_ref, hws_ref, disp_ref, bp_ref, gp_ref, bep_ref,
             wb_ref, out_ref):
        a = aggp_ref[...]
        disp = disp_ref[...]
        t = disp * (a[0, 0:NP, :] + a[1, 0:NP, :] + hws_ref[0:NP, :]) + bp_ref[...]
        h = _bn_packed(t, gp_ref[...], bep_ref[...], relu=True)
        out_ref[0:NP, :] = (
            jnp.dot(h, wb_ref[...], preferred_element_type=jnp.float32) * disp
        )
        out_ref[NP:NPP, :] = jnp.zeros((NPP - NP, PK * H), jnp.float32)

    return pl.pallas_call(
        body, out_shape=jax.ShapeDtypeStruct((NPP, PK * H), jnp.float32)
    )(aggp, hws, disp, bp, gp, bep, wblock)


def _final_call(aggp, hws, disp, bp, gp, bep):
    def body(aggp_ref, hws_ref, disp_ref, bp_ref, gp_ref, bep_ref, out_ref):
        a = aggp_ref[...]
        disp = disp_ref[...]
        t = disp * (a[0, 0:NP, :] + a[1, 0:NP, :] + hws_ref[0:NP, :]) + bp_ref[...]
        h = _bn_packed(t, gp_ref[...], bep_ref[...], relu=False)
        out_ref[0:NP, :] = h
        out_ref[NP:NPP, :] = jnp.zeros((NPP - NP, PK * H), jnp.float32)

    return pl.pallas_call(
        body, out_shape=jax.ShapeDtypeStruct((NPP, PK * H), jnp.float32)
    )(aggp, hws, disp, bp, gp, bep)


def _pad_idx(a, nw, epw, nchunk, fill):
    a = a.reshape(nw, epw)
    pad = jnp.full((nw, nchunk * C - epw), fill, jnp.int32)
    return jnp.concatenate([a, pad], axis=1).reshape(nw, nchunk, C)


def kernel(x, edge_index, batch, Ws, bs, gammas, betas):
    del batch
    E = edge_index.shape[1]
    epw = E // NW
    nchunk = (-(-epw // C) + 7) // 8 * 8
    src_r = _pad_idx(edge_index[0], NW, epw, nchunk, 0)
    dst_r = _pad_idx(edge_index[1], NW, epw, nchunk, NPAD - 1)

    ones32 = jnp.ones((C, DW), jnp.float32)
    zeros32 = jnp.zeros((RPS, H), jnp.float32)

    degp = _deg_call(dst_r, ones32, zeros32)
    degp_pk = jnp.reshape(degp, (NC, NPP, 128))
    disp, hws = _prelude_call(degp_pk, x, Ws[0])

    for l in range(L):
        tbl = jnp.reshape(hws, (NPAD, H))
        aggp = jnp.reshape(_agg_call(src_r, dst_r, tbl, zeros32),
                           (NC, NPP, PK * H))
        bp = jnp.tile(bs[l], PK).reshape(1, PK * H)
        gp = jnp.tile(gammas[l], PK).reshape(1, PK * H)
        bep = jnp.tile(betas[l], PK).reshape(1, PK * H)
        if l != L - 1:
            wblock = jnp.kron(jnp.eye(PK, dtype=jnp.float32), Ws[l + 1])
            hws = _layer_call(aggp, hws, disp, bp, gp, bep, wblock)
        else:
            outp = _final_call(aggp, hws, disp, bp, gp, bep)
    return jnp.reshape(outp, (NPAD, H))[0:N]
